# baseline (speedup 1.0000x reference)
import functools

import jax
import jax.numpy as jnp
from jax import lax
from jax.experimental import pallas as pl
from jax.experimental.pallas import tpu as pltpu
from jax.experimental.pallas import tpu_sc as plsc


def _sc_gather_body(table_hbm, idx_hbm, out_hbm, idx_v, rows_v, sem):
    info = plsc.get_sparse_core_info()
    wid = lax.axis_index("s") * info.num_cores + lax.axis_index("c")
    bpw = idx_v.shape[0]
    base = wid * bpw
    pltpu.sync_copy(idx_hbm.at[pl.ds(base, bpw)], idx_v)
    pltpu.async_copy(table_hbm.at[idx_v], rows_v, sem).wait()
    pltpu.sync_copy(rows_v, out_hbm.at[pl.ds(base, bpw)])


def _sc_time_emb(table, idx, n_rows, dim):
    info = plsc.get_sparse_core_info()
    nw = info.num_cores * info.num_subcores
    bpw = n_rows // nw
    mesh = plsc.VectorSubcoreMesh(core_axis_name="c", subcore_axis_name="s")
    k = functools.partial(
        pl.kernel,
        mesh=mesh,
        out_type=jax.ShapeDtypeStruct((n_rows, dim), jnp.float32),
        scratch_types=[
            pltpu.VMEM((bpw,), jnp.int32),
            pltpu.VMEM((bpw, dim), jnp.float32),
            pltpu.SemaphoreType.DMA,
        ],
    )(_sc_gather_body)
    return k(table, idx)


def _tc_add_body(emb_ref, x_ref, o_ref):
    o_ref[0] = x_ref[0] + emb_ref[0]


def kernel(x, hour, day, hour_table, day_table):
    B, N, T, D = x.shape
    TD = T * D
    dim_per = hour_table.shape[1]

    table = jnp.concatenate([hour_table, day_table], axis=0)
    table = jnp.pad(table, ((0, 0), (0, 128 - dim_per)))
    idx = jnp.stack(
        [hour.astype(jnp.int32),
         day.astype(jnp.int32) + hour_table.shape[0]],
        axis=-1,
    ).reshape(-1)

    rows = _sc_time_emb(table, idx, 2 * B * T, 128)
    emb = rows[:, :dim_per].reshape(B, 1, T * 2 * dim_per)

    x3 = x.reshape(B, N, TD)
    out = pl.pallas_call(
        _tc_add_body,
        grid=(B,),
        in_specs=[
            pl.BlockSpec((1, 1, TD), lambda b: (b, 0, 0)),
            pl.BlockSpec((1, N, TD), lambda b: (b, 0, 0)),
        ],
        out_specs=pl.BlockSpec((1, N, TD), lambda b: (b, 0, 0)),
        out_shape=jax.ShapeDtypeStruct((B, N, TD), x.dtype),
    )(emb, x3)
    return out.reshape(B, N, T, D)


# speedup vs baseline: 1.1256x; 1.1256x over previous
import functools

import jax
import jax.numpy as jnp
from jax import lax
from jax.experimental import pallas as pl
from jax.experimental.pallas import tpu as pltpu
from jax.experimental.pallas import tpu_sc as plsc

BB = 4


def _sc_gather_body(table_hbm, idx_hbm, out_hbm, idx_v, rows_v, sem):
    info = plsc.get_sparse_core_info()
    wid = lax.axis_index("s") * info.num_cores + lax.axis_index("c")
    bpw = idx_v.shape[0]
    base = wid * bpw
    pltpu.sync_copy(idx_hbm.at[pl.ds(base, bpw)], idx_v)
    pltpu.async_copy(table_hbm.at[idx_v], rows_v, sem).wait()
    pltpu.sync_copy(rows_v, out_hbm.at[pl.ds(base, bpw)])


def _sc_time_emb(table, idx, n_rows, dim):
    info = plsc.get_sparse_core_info()
    nw = info.num_cores * info.num_subcores
    bpw = n_rows // nw
    mesh = plsc.VectorSubcoreMesh(core_axis_name="c", subcore_axis_name="s")
    k = functools.partial(
        pl.kernel,
        mesh=mesh,
        out_type=jax.ShapeDtypeStruct((n_rows, dim), jnp.float32),
        scratch_types=[
            pltpu.VMEM((bpw,), jnp.int32),
            pltpu.VMEM((bpw, dim), jnp.float32),
            pltpu.SemaphoreType.DMA,
        ],
    )(_sc_gather_body)
    return k(table, idx)


def _tc_add_body(rows_ref, x_ref, o_ref):
    n_rows = rows_ref.shape[1]
    dim_per = 32
    embs = []
    for i in range(BB):
        a = rows_ref[i]
        parts = [a[r:r + 1, :dim_per] for r in range(n_rows)]
        embs.append(jnp.concatenate(parts, axis=1)[None])
    emb = jnp.concatenate(embs, axis=0)
    o_ref[...] = x_ref[...] + emb


def kernel(x, hour, day, hour_table, day_table):
    B, N, T, D = x.shape
    TD = T * D
    dim_per = hour_table.shape[1]

    table = jnp.concatenate([hour_table, day_table], axis=0)
    table = jnp.pad(table, ((0, 0), (0, 128 - dim_per)))
    idx = jnp.stack(
        [hour.astype(jnp.int32),
         day.astype(jnp.int32) + hour_table.shape[0]],
        axis=-1,
    ).reshape(-1)

    rows = _sc_time_emb(table, idx, 2 * B * T, 128)
    rows3 = rows.reshape(B, 2 * T, 128)

    x3 = x.reshape(B, N, TD)
    out = pl.pallas_call(
        _tc_add_body,
        grid=(B // BB,),
        in_specs=[
            pl.BlockSpec((BB, 2 * T, 128), lambda b: (b, 0, 0)),
            pl.BlockSpec((BB, N, TD), lambda b: (b, 0, 0)),
        ],
        out_specs=pl.BlockSpec((BB, N, TD), lambda b: (b, 0, 0)),
        out_shape=jax.ShapeDtypeStruct((B, N, TD), x.dtype),
    )(rows3, x3)
    return out.reshape(B, N, T, D)


# speedup vs baseline: 1.1340x; 1.0075x over previous
import functools

import jax
import jax.numpy as jnp
from jax import lax
from jax.experimental import pallas as pl
from jax.experimental.pallas import tpu as pltpu
from jax.experimental.pallas import tpu_sc as plsc

BB = 8


def _sc_gather_body(table_hbm, idx_hbm, out_hbm, idx_v, rows_v, sem):
    info = plsc.get_sparse_core_info()
    wid = lax.axis_index("s") * info.num_cores + lax.axis_index("c")
    bpw = idx_v.shape[0]
    base = wid * bpw
    pltpu.sync_copy(idx_hbm.at[pl.ds(base, bpw)], idx_v)
    pltpu.async_copy(table_hbm.at[idx_v], rows_v, sem).wait()
    pltpu.sync_copy(rows_v, out_hbm.at[pl.ds(base, bpw)])


def _sc_time_emb(table, idx, n_rows, dim):
    info = plsc.get_sparse_core_info()
    nw = info.num_cores * info.num_subcores
    bpw = n_rows // nw
    mesh = plsc.VectorSubcoreMesh(core_axis_name="c", subcore_axis_name="s")
    k = functools.partial(
        pl.kernel,
        mesh=mesh,
        out_type=jax.ShapeDtypeStruct((n_rows, dim), jnp.float32),
        scratch_types=[
            pltpu.VMEM((bpw,), jnp.int32),
            pltpu.VMEM((bpw, dim), jnp.float32),
            pltpu.SemaphoreType.DMA,
        ],
    )(_sc_gather_body)
    return k(table, idx)


def _tc_add_body(rows_ref, x_ref, o_ref):
    n_rows = rows_ref.shape[1]
    dim_per = 32
    embs = []
    for i in range(BB):
        a = rows_ref[i]
        parts = [a[r:r + 1, :dim_per] for r in range(n_rows)]
        embs.append(jnp.concatenate(parts, axis=1)[None])
    emb = jnp.concatenate(embs, axis=0)
    o_ref[...] = x_ref[...] + emb


def kernel(x, hour, day, hour_table, day_table):
    B, N, T, D = x.shape
    TD = T * D
    dim_per = hour_table.shape[1]

    table = jnp.concatenate([hour_table, day_table], axis=0)
    table = jnp.pad(table, ((0, 0), (0, 128 - dim_per)))
    idx = jnp.stack(
        [hour.astype(jnp.int32),
         day.astype(jnp.int32) + hour_table.shape[0]],
        axis=-1,
    ).reshape(-1)

    rows = _sc_time_emb(table, idx, 2 * B * T, 128)
    rows3 = rows.reshape(B, 2 * T, 128)

    x3 = x.reshape(B, N, TD)
    out = pl.pallas_call(
        _tc_add_body,
        grid=(B // BB,),
        in_specs=[
            pl.BlockSpec((BB, 2 * T, 128), lambda b: (b, 0, 0)),
            pl.BlockSpec((BB, N, TD), lambda b: (b, 0, 0)),
        ],
        out_specs=pl.BlockSpec((BB, N, TD), lambda b: (b, 0, 0)),
        out_shape=jax.ShapeDtypeStruct((B, N, TD), x.dtype),
    )(rows3, x3)
    return out.reshape(B, N, T, D)
